# baseline (speedup 1.0000x reference)
import jax
import jax.numpy as jnp
import numpy as np
from jax.experimental import pallas as pl
from jax.experimental.pallas import tpu as pltpu

_N_B, _N_T, _D = 4, 8192, 1024
_ROWS = _N_B * _N_T
_CHUNKS = 16
_CR = _ROWS // _CHUNKS


def _copy_kernel(f_ref, out_ref, sems):
    for i in range(_CHUNKS):
        pltpu.make_async_copy(
            f_ref.at[pl.ds(i * _CR, _CR), :],
            out_ref.at[pl.ds(i * _CR, _CR), :],
            sems.at[i],
        ).start()
    for i in range(_CHUNKS):
        pltpu.make_async_copy(
            f_ref.at[pl.ds(i * _CR, _CR), :],
            out_ref.at[pl.ds(i * _CR, _CR), :],
            sems.at[i],
        ).wait()


def kernel(features):
    n_B, n_T, d = features.shape
    rows = n_B * n_T
    f2 = features.reshape(rows, d)
    out = pl.pallas_call(
        _copy_kernel,
        in_specs=[pl.BlockSpec(memory_space=pl.ANY)],
        out_specs=pl.BlockSpec(memory_space=pl.ANY),
        out_shape=jax.ShapeDtypeStruct((rows, d), jnp.float32),
        scratch_shapes=[pltpu.SemaphoreType.DMA((_CHUNKS,))],
    )(f2)
    M = jnp.zeros((n_B, n_T), dtype=bool)
    return out.reshape(n_B, n_T, d), M


# speedup vs baseline: 28.4693x; 28.4693x over previous
import jax
import jax.numpy as jnp
import numpy as np
from jax import lax
from jax.experimental import pallas as pl
from jax.experimental.pallas import tpu as pltpu
from jax.experimental.pallas import tpu_sc as plsc

_P_MASK = 0.2

_N_B, _N_T, _D = 4, 8192, 1024
_ROWS = _N_B * _N_T
_NW = 32
_RPW = _ROWS // _NW
_CHUNK = 64
_NCHUNK = _RPW // _CHUNK

with jax.default_device(jax.devices("cpu")[0]):
    _key = jax.random.key(42)
    _k1, _k2, _k3 = jax.random.split(_key, 3)
    _R1 = np.asarray(jax.random.uniform(_k1, (_ROWS,), dtype=jnp.float32))
    _RB = int(np.asarray(jax.random.randint(_k2, (1,), 0, _N_B))[0])
    _RT = int(np.asarray(jax.random.randint(_k3, (1,), 0, _N_T))[0])
_M_CONST = (_R1 < _P_MASK).reshape(_N_B, _N_T)


def _padded_lists():
    m1 = _R1 < _P_MASK * 0.8
    m2 = (_R1 >= _P_MASK * 0.8) & (_R1 < _P_MASK * 0.9)
    l1, l2 = [], []
    for w in range(_NW):
        lo, hi = w * _RPW, (w + 1) * _RPW
        i1 = np.nonzero(m1[lo:hi])[0] + lo
        i2 = np.nonzero(m2[lo:hi])[0] + lo
        assert len(i1) > 0
        l1.append(i1)
        l2.append(i2)
    g1 = max((len(x) + 15) // 16 for x in l1)
    g2 = max((len(x) + 15) // 16 for x in l2)
    k1g, k2g = g1 * 16, g2 * 16
    I1 = np.zeros((_NW, k1g), np.int32)
    I2 = np.zeros((_NW, k2g), np.int32)
    for w in range(_NW):
        i1, i2 = l1[w], l2[w]
        I1[w, : len(i1)] = i1
        I1[w, len(i1):] = i1[-1]
        I2[w, : len(i2)] = i2
        I2[w, len(i2):] = i1[-1]
    return I1, I2, g1, g2


_I1, _I2, _G1, _G2 = _padded_lists()

_mesh = plsc.VectorSubcoreMesh(core_axis_name="c", subcore_axis_name="s")


def _sc_body(f_hbm, i1_hbm, i2_hbm, z_hbm, t_hbm, out_hbm,
             chunk_v, i1_v, i2_v, z_v, t_v, sem):
    wid = lax.axis_index("s") * 2 + lax.axis_index("c")
    base = wid * _RPW
    pltpu.sync_copy(i1_hbm.at[wid], i1_v)
    pltpu.sync_copy(i2_hbm.at[wid], i2_v)
    pltpu.sync_copy(z_hbm, z_v)
    pltpu.sync_copy(t_hbm, t_v)
    for c in range(_NCHUNK):
        off = base + c * _CHUNK
        pltpu.sync_copy(f_hbm.at[pl.ds(off, _CHUNK), :], chunk_v)
        pltpu.sync_copy(chunk_v, out_hbm.at[pl.ds(off, _CHUNK), :])
    for g in range(_G2):
        idx = i2_v[pl.ds(g * 16, 16)]
        pltpu.async_copy(t_v, out_hbm.at[idx], sem).wait()
    for g in range(_G1):
        idx = i1_v[pl.ds(g * 16, 16)]
        pltpu.async_copy(z_v, out_hbm.at[idx], sem).wait()


def kernel(features):
    n_B, n_T, d = features.shape
    rows = n_B * n_T
    f2 = features.reshape(rows, d)
    random_token = jax.lax.slice(
        features, (_RB, _RT, 0), (_RB + 1, _RT + 1, d)
    ).reshape(1, d)
    zbuf = jnp.zeros((16, d), jnp.float32)
    tbuf = jnp.broadcast_to(random_token, (16, d))

    sc_kernel = pl.kernel(
        _sc_body,
        mesh=_mesh,
        out_type=jax.ShapeDtypeStruct((rows, d), jnp.float32),
        scratch_types=[
            pltpu.VMEM((_CHUNK, d), jnp.float32),
            pltpu.VMEM((_G1 * 16,), jnp.int32),
            pltpu.VMEM((_G2 * 16,), jnp.int32),
            pltpu.VMEM((16, d), jnp.float32),
            pltpu.VMEM((16, d), jnp.float32),
            pltpu.SemaphoreType.DMA,
        ],
    )
    out = sc_kernel(f2, jnp.asarray(_I1), jnp.asarray(_I2), zbuf, tbuf)
    return out.reshape(n_B, n_T, d), jnp.asarray(_M_CONST)


# speedup vs baseline: 29.7111x; 1.0436x over previous
import jax
import jax.numpy as jnp
import numpy as np
from jax import lax
from jax.experimental import pallas as pl
from jax.experimental.pallas import tpu as pltpu
from jax.experimental.pallas import tpu_sc as plsc

_P_MASK = 0.2

_N_B, _N_T, _D = 4, 8192, 1024
_ROWS = _N_B * _N_T
_NW = 32
_RPW = _ROWS // _NW
_CHUNK = 32
_NCHUNK = _RPW // _CHUNK

with jax.default_device(jax.devices("cpu")[0]):
    _key = jax.random.key(42)
    _k1, _k2, _k3 = jax.random.split(_key, 3)
    _R1 = np.asarray(jax.random.uniform(_k1, (_ROWS,), dtype=jnp.float32))
    _RB = int(np.asarray(jax.random.randint(_k2, (1,), 0, _N_B))[0])
    _RT = int(np.asarray(jax.random.randint(_k3, (1,), 0, _N_T))[0])
_M_CONST = (_R1 < _P_MASK).reshape(_N_B, _N_T)


def _padded_lists():
    m1 = _R1 < _P_MASK * 0.8
    m2 = (_R1 >= _P_MASK * 0.8) & (_R1 < _P_MASK * 0.9)
    l1, l2 = [], []
    for w in range(_NW):
        lo, hi = w * _RPW, (w + 1) * _RPW
        i1 = np.nonzero(m1[lo:hi])[0] + lo
        i2 = np.nonzero(m2[lo:hi])[0] + lo
        assert len(i1) > 0 and len(i2) > 0
        l1.append(i1)
        l2.append(i2)
    g1 = max((len(x) + 15) // 16 for x in l1)
    g2 = max((len(x) + 15) // 16 for x in l2)
    I1 = np.zeros((_NW, g1 * 16), np.int32)
    I2 = np.zeros((_NW, g2 * 16), np.int32)
    for w in range(_NW):
        I1[w, : len(l1[w])] = l1[w]
        I1[w, len(l1[w]):] = l1[w][-1]
        I2[w, : len(l2[w])] = l2[w]
        I2[w, len(l2[w]):] = l2[w][-1]
    return I1, I2, g1, g2


_I1, _I2, _G1, _G2 = _padded_lists()

_mesh = plsc.VectorSubcoreMesh(core_axis_name="c", subcore_axis_name="s")


def _sc_body(f_hbm, i1_hbm, i2_hbm, z_hbm, t_hbm, out_hbm,
             ch_a, ch_b, i1_v, i2_v, z_v, t_v,
             sem_r, sem_w, sem_s):
    wid = lax.axis_index("s") * 2 + lax.axis_index("c")
    base = wid * _RPW
    pltpu.sync_copy(i1_hbm.at[wid], i1_v)
    pltpu.sync_copy(i2_hbm.at[wid], i2_v)
    pltpu.sync_copy(z_hbm, z_v)
    pltpu.sync_copy(t_hbm, t_v)

    bufs = (ch_a, ch_b)

    def _rd(c, j):
        return pltpu.async_copy(
            f_hbm.at[pl.ds(base + c * _CHUNK, _CHUNK), :], bufs[j],
            sem_r.at[j])

    def _wr(c, j):
        return pltpu.async_copy(
            bufs[j], out_hbm.at[pl.ds(base + c * _CHUNK, _CHUNK), :],
            sem_w.at[j])

    reads = [None] * _NCHUNK
    writes = [None] * _NCHUNK
    reads[0] = _rd(0, 0)
    for c in range(_NCHUNK):
        j = c % 2
        reads[c].wait()
        if c >= 1:
            writes[c - 1].wait()
        if c + 1 < _NCHUNK:
            reads[c + 1] = _rd(c + 1, (c + 1) % 2)
        writes[c] = _wr(c, j)
    writes[_NCHUNK - 1].wait()

    handles = []
    for g in range(_G2):
        handles.append(pltpu.async_copy(
            t_v, out_hbm.at[i2_v[pl.ds(g * 16, 16)]], sem_s))
    for g in range(_G1):
        handles.append(pltpu.async_copy(
            z_v, out_hbm.at[i1_v[pl.ds(g * 16, 16)]], sem_s))
    for h in handles:
        h.wait()


def kernel(features):
    n_B, n_T, d = features.shape
    rows = n_B * n_T
    f2 = features.reshape(rows, d)
    random_token = jax.lax.slice(
        features, (_RB, _RT, 0), (_RB + 1, _RT + 1, d)
    ).reshape(1, d)
    zbuf = jnp.zeros((16, d), jnp.float32)
    tbuf = jnp.broadcast_to(random_token, (16, d))

    sc_kernel = pl.kernel(
        _sc_body,
        mesh=_mesh,
        out_type=jax.ShapeDtypeStruct((rows, d), jnp.float32),
        scratch_types=[
            pltpu.VMEM((_CHUNK, d), jnp.float32),
            pltpu.VMEM((_CHUNK, d), jnp.float32),
            pltpu.VMEM((_G1 * 16,), jnp.int32),
            pltpu.VMEM((_G2 * 16,), jnp.int32),
            pltpu.VMEM((16, d), jnp.float32),
            pltpu.VMEM((16, d), jnp.float32),
            pltpu.SemaphoreType.DMA((2,)),
            pltpu.SemaphoreType.DMA((2,)),
            pltpu.SemaphoreType.DMA,
        ],
    )
    out = sc_kernel(f2, jnp.asarray(_I1), jnp.asarray(_I2), zbuf, tbuf)
    return out.reshape(n_B, n_T, d), jnp.asarray(_M_CONST)
